# baseline (speedup 1.0000x reference)
import functools

import jax
import jax.numpy as jnp
from jax import lax
from jax.experimental import pallas as pl
from jax.experimental.pallas import tpu as pltpu
from jax.experimental.pallas import tpu_sc as plsc

B, S, H, Q = 4096, 100, 128, 128
R = S * B
EPS = 1e-5
NW = 32
CH = 256
NCHUNK = R // (NW * CH)

_sc_mesh = plsc.VectorSubcoreMesh(core_axis_name="c", subcore_axis_name="s")


@functools.partial(
    pl.kernel, mesh=_sc_mesh,
    out_type=jax.ShapeDtypeStruct((R, H), jnp.float32),
    scratch_types=[
        pltpu.VMEM((CH,), jnp.int32),
        pltpu.VMEM((CH, H), jnp.float32),
        pltpu.SemaphoreType.DMA,
    ],
)
def _sc_date_gather(table_hbm, idx_hbm, out_hbm, idx_v, rows_v, sem):
    wid = lax.axis_index("s") * 2 + lax.axis_index("c")

    def body(i, _):
        base = (wid * NCHUNK + i) * CH
        pltpu.sync_copy(idx_hbm.at[pl.ds(base, CH)], idx_v)
        pltpu.async_copy(table_hbm.at[idx_v], rows_v, sem).wait()
        pltpu.sync_copy(rows_v, out_hbm.at[pl.ds(base, CH)])
        return 0

    lax.fori_loop(0, NCHUNK, body, 0)


def _fused_body(floor_ref, delta_ref, tv_ref, text_ref, demb_ref, colp_ref,
                wcol_ref, wc_ref, rhs_ref, bias_ref, gamma_ref, beta_ref,
                out_ref):
    f32 = jnp.float32
    bf16 = jnp.bfloat16
    s = pl.program_id(0)

    fl = floor_ref[...][0]
    d = delta_ref[...][0]
    tvl = tv_ref[...]

    qi = jax.lax.broadcasted_iota(jnp.int32, (128, B), 0)
    bc = lambda x: jnp.broadcast_to(x, (128, B))

    ohn = (qi == bc(fl)).astype(bf16)
    ohnd = ohn * bc(d.astype(bf16))
    tv_sel = jnp.where(s == S - 1, tvl, -1)
    oht = (qi == bc(tv_sel)).astype(bf16)
    lhs_t = jnp.concatenate([ohn, ohnd, oht], axis=0)

    txt = jnp.where(s == S - 1, 0.0, text_ref[...][0]).astype(bf16)

    acc = jnp.dot(txt, wc_ref[...], preferred_element_type=f32)
    acc = acc + jax.lax.dot_general(
        lhs_t, rhs_ref[...], dimension_numbers=(((0,), (0,)), ((), ())),
        preferred_element_type=f32)
    acc = acc + demb_ref[...][0]

    cb = (jnp.dot(colp_ref[...], wcol_ref[...],
                  preferred_element_type=f32) + bias_ref[...])
    ohrow = (jax.lax.broadcasted_iota(jnp.int32, (1, 128), 1) == s)
    crow = jnp.dot(ohrow.astype(f32), cb, preferred_element_type=f32)
    acc = acc + crow

    m = jnp.mean(acc, axis=1, keepdims=True)
    c = acc - m
    v = jnp.mean(c * c, axis=1, keepdims=True)
    y = c * jax.lax.rsqrt(v + EPS) * gamma_ref[...] + beta_ref[...]
    out_ref[...] = y[None]


@jax.jit
def _run(floor_t, delta_t, tv, text_t, demb_t, colp, wcol,
         wc, rhs_static, bias, gamma, beta):
    vec_spec = pl.BlockSpec((1, 1, B), lambda s: (s, 0, 0))
    full_spec = lambda r, c: pl.BlockSpec((r, c), lambda s: (0, 0))
    return pl.pallas_call(
        _fused_body,
        grid=(S,),
        in_specs=[
            vec_spec,
            vec_spec,
            full_spec(1, B),
            pl.BlockSpec((1, B, H), lambda s: (s, 0, 0)),
            pl.BlockSpec((1, B, H), lambda s: (s, 0, 0)),
            full_spec(128, 128),
            full_spec(128, 128),
            full_spec(128, 128),
            full_spec(384, 128),
            full_spec(1, 128),
            full_spec(1, 128),
            full_spec(1, 128),
        ],
        out_specs=pl.BlockSpec((1, B, H), lambda s: (s, 0, 0)),
        out_shape=jax.ShapeDtypeStruct((S, B, H), jnp.float32),
    )(floor_t, delta_t, tv, text_t, demb_t, colp, wcol,
      wc, rhs_static, bias, gamma, beta)


def kernel(number_percentile_floor, number_percentile_delta,
           date_year_month_day_weekday, column_embeddings, text_embeddings,
           target, target_delta, is_regression, number_emb,
           target_classif_emb, year_emb, month_emb, day_emb, weekday_emb,
           W_col, b_col, W_content, b_content, ln_gamma, ln_beta):
    bf16 = jnp.bfloat16
    floor = number_percentile_floor.astype(jnp.int32)
    date = date_year_month_day_weekday.astype(jnp.int32)
    floor_t = floor.T.reshape(S, 1, B)
    delta_t = number_percentile_delta.T.reshape(S, 1, B)
    codes = (date[:, :, 0] + 8 * date[:, :, 1] + 64 * date[:, :, 2]
             + 512 * date[:, :, 3]).T.reshape(R)
    tgt = target.astype(jnp.int32)
    tv = jnp.where(tgt < 0, 0, tgt + 1).reshape(1, B)
    text_t = jnp.transpose(text_embeddings, (1, 0, 2))
    colp = jnp.concatenate(
        [column_embeddings, jnp.zeros((128 - S, H), jnp.float32)],
        axis=0).astype(bf16)
    ndiff = jnp.concatenate([number_emb[1:], number_emb[-1:]],
                            axis=0) - number_emb
    y8, m8, d8, w8 = year_emb[:8], month_emb[:8], day_emb[:8], weekday_emb[:8]
    t01 = (m8[:, None, :] + y8[None, :, :]).reshape(64, H)
    t23 = (w8[:, None, :] + d8[None, :, :]).reshape(64, H)
    table4 = (t23[:, None, :] + t01[None, :, :]).reshape(4096, H)
    rhs_static = jnp.concatenate(
        [number_emb, ndiff, target_classif_emb], axis=0).astype(bf16)
    bias = (b_col + b_content).reshape(1, H)
    demb = _sc_date_gather(table4, codes)
    demb_t = demb.reshape(S, B, H)
    out_t = _run(floor_t, delta_t, tv, text_t, demb_t, colp,
                 W_col.astype(bf16), W_content.astype(bf16),
                 rhs_static, bias, ln_gamma.reshape(1, H),
                 ln_beta.reshape(1, H))
    return jnp.transpose(out_t, (1, 0, 2))


# speedup vs baseline: 1.5591x; 1.5591x over previous
import functools

import jax
import jax.numpy as jnp
from jax.experimental import pallas as pl

B, S, H, Q = 4096, 100, 128, 128
EPS = 1e-5


def _fused_body(floor_ref, delta_ref, c01_ref, c23_ref, tv_ref,
                text_ref, colp_ref, wcol_ref, wc_ref, rhs_ref, bias_ref,
                gamma_ref, beta_ref, out_ref):
    f32 = jnp.float32
    bf16 = jnp.bfloat16
    s = pl.program_id(0)

    fl = floor_ref[...][0]
    d = delta_ref[...][0]
    tvl = tv_ref[...]

    qi = jax.lax.broadcasted_iota(jnp.int32, (128, B), 0)
    bc = lambda x: jnp.broadcast_to(x, (128, B))

    ohn = (qi == bc(fl)).astype(bf16)
    ohnd = ohn * bc(d.astype(bf16))
    mh = ((qi == bc(c01_ref[...][0]))
          | (qi == bc(c23_ref[...][0]))).astype(bf16)
    tv_sel = jnp.where(s == S - 1, tvl, -1)
    oht = (qi == bc(tv_sel)).astype(bf16)

    lhs_t = jnp.concatenate([ohn, ohnd, mh, oht], axis=0)

    txt = jnp.where(s == S - 1, 0.0, text_ref[...][0]).astype(bf16)

    acc = jnp.dot(txt, wc_ref[...], preferred_element_type=f32)
    acc = acc + jax.lax.dot_general(
        lhs_t, rhs_ref[...], dimension_numbers=(((0,), (0,)), ((), ())),
        preferred_element_type=f32)

    cb = (jnp.dot(colp_ref[...], wcol_ref[...],
                  preferred_element_type=f32) + bias_ref[...])
    ohrow = (jax.lax.broadcasted_iota(jnp.int32, (1, 128), 1) == s)
    crow = jnp.dot(ohrow.astype(f32), cb, preferred_element_type=f32)
    acc = acc + crow

    m = jnp.mean(acc, axis=1, keepdims=True)
    c = acc - m
    v = jnp.mean(c * c, axis=1, keepdims=True)
    y = c * jax.lax.rsqrt(v + EPS) * gamma_ref[...] + beta_ref[...]
    out_ref[...] = y[None]


@jax.jit
def _run(floor_t, delta_t, c01_t, c23_t, tv, text_t, colp, wcol,
         wc, rhs_static, bias, gamma, beta):
    vec_spec = pl.BlockSpec((1, 1, B), lambda s: (s, 0, 0))
    full_spec = lambda r, c: pl.BlockSpec((r, c), lambda s: (0, 0))
    return pl.pallas_call(
        _fused_body,
        grid=(S,),
        in_specs=[
            vec_spec,
            vec_spec,
            vec_spec,
            vec_spec,
            full_spec(1, B),
            pl.BlockSpec((1, B, H), lambda s: (s, 0, 0)),
            full_spec(128, 128),
            full_spec(128, 128),
            full_spec(128, 128),
            full_spec(512, 128),
            full_spec(1, 128),
            full_spec(1, 128),
            full_spec(1, 128),
        ],
        out_specs=pl.BlockSpec((1, B, H), lambda s: (s, 0, 0)),
        out_shape=jax.ShapeDtypeStruct((S, B, H), jnp.float32),
    )(floor_t, delta_t, c01_t, c23_t, tv, text_t, colp, wcol,
      wc, rhs_static, bias, gamma, beta)


def kernel(number_percentile_floor, number_percentile_delta,
           date_year_month_day_weekday, column_embeddings, text_embeddings,
           target, target_delta, is_regression, number_emb,
           target_classif_emb, year_emb, month_emb, day_emb, weekday_emb,
           W_col, b_col, W_content, b_content, ln_gamma, ln_beta):
    bf16 = jnp.bfloat16
    floor = number_percentile_floor.astype(jnp.int32)
    date = date_year_month_day_weekday.astype(jnp.int32)
    floor_t = floor.T.reshape(S, 1, B)
    delta_t = number_percentile_delta.T.reshape(S, 1, B)
    c01_t = (date[:, :, 0] + 8 * date[:, :, 1]).T.reshape(S, 1, B)
    c23_t = (64 + date[:, :, 2] + 8 * date[:, :, 3]).T.reshape(S, 1, B)
    tgt = target.astype(jnp.int32)
    tv = jnp.where(tgt < 0, 0, tgt + 1).reshape(1, B)
    text_t = jnp.transpose(text_embeddings, (1, 0, 2))
    colp = jnp.concatenate(
        [column_embeddings, jnp.zeros((128 - S, H), jnp.float32)],
        axis=0).astype(bf16)
    ndiff = jnp.concatenate([number_emb[1:], number_emb[-1:]],
                            axis=0) - number_emb
    tab01 = (month_emb[:8][:, None, :] + year_emb[:8][None, :, :])
    tab23 = (weekday_emb[:8][:, None, :] + day_emb[:8][None, :, :])
    rhs_static = jnp.concatenate(
        [number_emb, ndiff, tab01.reshape(64, H), tab23.reshape(64, H),
         target_classif_emb], axis=0).astype(bf16)
    bias = (b_col + b_content).reshape(1, H)
    out_t = _run(floor_t, delta_t, c01_t, c23_t, tv,
                 text_t, colp, W_col.astype(bf16), W_content.astype(bf16),
                 rhs_static, bias, ln_gamma.reshape(1, H),
                 ln_beta.reshape(1, H))
    return jnp.transpose(out_t, (1, 0, 2))
